# baseline (speedup 1.0000x reference)
import functools

import jax
import jax.numpy as jnp
from jax import lax
from jax.experimental import pallas as pl
from jax.experimental.pallas import tpu as pltpu
from jax.experimental.pallas import tpu_sc as plsc

NC = 2
NS = 16
L = 16

CH = 2048


@functools.lru_cache(maxsize=None)
def _build_sc_kernel(M, B, D_IN, D_OUT):
    n_slot = (M + 127) // 128 * 128
    rows_per_core = B // NC
    rows_per_tile = B // (NC * NS)
    n_vidx = rows_per_tile + L

    mesh = plsc.VectorSubcoreMesh(
        core_axis_name="c", subcore_axis_name="s",
        num_cores=NC, num_subcores=NS)

    @functools.partial(
        pl.kernel,
        out_type=(
            jax.ShapeDtypeStruct((B, D_IN), jnp.float32),
            jax.ShapeDtypeStruct((B, D_OUT), jnp.float32),
        ),
        mesh=mesh,
        compiler_params=pltpu.CompilerParams(
            needs_layout_passes=False, use_tc_tiling_on_sc=False),
        scratch_types=[
            pltpu.VMEM_SHARED((rows_per_core,), jnp.int32),
        ],
    )
    def sc_kernel(x_hbm, y_hbm, widx_hbm, ridx_hbm, outx_hbm, outy_hbm,
                  srows_sh):
        cid = lax.axis_index("c")
        sid = lax.axis_index("s")

        @pl.when(sid == 0)
        def _build():
            def build(slot, wbuf, rbuf, robuf):
                iota = jnp.arange(L, dtype=jnp.int32)
                zero16 = jnp.zeros((L,), jnp.int32)

                with jax.named_scope("ph_memset"):
                    def zloop(i, carry):
                        slot[pl.ds(i * L, L)] = zero16
                        return carry
                    lax.fori_loop(0, n_slot // L, zloop, 0)

                def wchunk(ck, carry):
                    pltpu.sync_copy(widx_hbm.at[pl.ds(ck * CH, CH)], wbuf)

                    def scat(i, c2):
                        idxv = wbuf[pl.ds(i * L, L)]
                        jv = iota + (ck * CH + i * L + 1)
                        plsc.store_scatter(slot, [idxv], jv)
                        return c2
                    lax.fori_loop(0, CH // L, scat, 0)

                    def rep(i, c2):
                        idxv = wbuf[pl.ds(i * L, L)]
                        jv = iota + (ck * CH + i * L + 1)
                        m = plsc.load_gather(slot, [idxv]) < jv

                        def wbody(mm):
                            plsc.store_scatter(slot, [idxv], jv, mask=mm)
                            return plsc.load_gather(slot, [idxv]) < jv
                        lax.while_loop(lambda mm: jnp.any(mm), wbody, m)
                        return c2
                    lax.fori_loop(0, CH // L, rep, 0)
                    return carry
                with jax.named_scope("ph_scatter"):
                    lax.fori_loop(0, B // CH, wchunk, 0)

                rbase = cid * rows_per_core

                def rchunk(ck, carry):
                    pltpu.sync_copy(
                        ridx_hbm.at[pl.ds(rbase + ck * CH, CH)], rbuf)

                    def g(i, c2):
                        idxv = rbuf[pl.ds(i * L, L)]
                        robuf[pl.ds(i * L, L)] = plsc.load_gather(
                            slot, [idxv])
                        return c2
                    lax.fori_loop(0, CH // L, g, 0)
                    pltpu.sync_copy(robuf, srows_sh.at[pl.ds(ck * CH, CH)])
                    return carry
                with jax.named_scope("ph_lookup"):
                    lax.fori_loop(0, rows_per_core // CH, rchunk, 0)

            pl.run_scoped(
                build,
                pltpu.VMEM((n_slot,), jnp.int32),
                pltpu.VMEM((CH,), jnp.int32),
                pltpu.VMEM((CH,), jnp.int32),
                pltpu.VMEM((CH,), jnp.int32),
            )

        plsc.subcore_barrier()

        def phasec(sv, vidx, rnk, xbuf, ybuf, sem):
            base = sid * rows_per_tile
            gbase = cid * rows_per_core + base
            pltpu.sync_copy(srows_sh.at[pl.ds(base, rows_per_tile)], sv)

            zero16 = jnp.zeros((L,), jnp.int32)
            for i in range(n_vidx // L):
                vidx[pl.ds(i * L, L)] = zero16

            def comp(i, off):
                s16 = sv[pl.ds(i * L, L)]
                m = s16 > 0
                mi = jnp.where(m, 1, 0).astype(jnp.int32)
                incl = plsc.cumsum(mi)
                rankv = off + incl - 1
                rnk[pl.ds(i * L, L)] = jnp.where(m, rankv, -1)
                plsc.store_scatter(vidx, [rankv], s16 - 1, mask=m)
                return off + jnp.max(incl)
            off = lax.fori_loop(
                0, rows_per_tile // L, comp,
                jnp.zeros((L,), jnp.int32))
            nv = off[0]

            nch = (nv + L - 1) // L

            def gch(k, carry):
                idxv = vidx[pl.ds(k * L, L)]
                cpx = pltpu.async_copy(
                    x_hbm.at[idxv], xbuf.at[pl.ds(k * L, L)], sem)
                cpy = pltpu.async_copy(
                    y_hbm.at[idxv], ybuf.at[pl.ds(k * L, L)], sem)
                cpx.wait()
                cpy.wait()
                return carry
            with jax.named_scope("ph_gather"):
                lax.fori_loop(0, nch, gch, 0)

            zx = jnp.zeros((L,), jnp.float32)

            def mv(t, carry):
                lp = rows_per_tile - 1 - t
                rv = plsc.load_gather(
                    rnk, [jnp.full((L,), lp, jnp.int32)])[0]

                @pl.when(rv >= 0)
                def _():
                    for v in range(D_IN // L):
                        xbuf[lp, pl.ds(v * L, L)] = xbuf[rv, pl.ds(v * L, L)]
                    for v in range(D_OUT // L):
                        ybuf[lp, pl.ds(v * L, L)] = ybuf[rv, pl.ds(v * L, L)]

                @pl.when(rv < 0)
                def _():
                    for v in range(D_IN // L):
                        xbuf[lp, pl.ds(v * L, L)] = zx
                    for v in range(D_OUT // L):
                        ybuf[lp, pl.ds(v * L, L)] = zx
                return carry
            with jax.named_scope("ph_move"):
                lax.fori_loop(0, rows_per_tile, mv, 0)

            pltpu.sync_copy(xbuf, outx_hbm.at[pl.ds(gbase, rows_per_tile)])
            pltpu.sync_copy(ybuf, outy_hbm.at[pl.ds(gbase, rows_per_tile)])

        pl.run_scoped(
            phasec,
            pltpu.VMEM((rows_per_tile,), jnp.int32),
            pltpu.VMEM((n_vidx,), jnp.int32),
            pltpu.VMEM((rows_per_tile + L,), jnp.int32),
            pltpu.VMEM((rows_per_tile, D_IN), jnp.float32),
            pltpu.VMEM((rows_per_tile, D_OUT), jnp.float32),
            pltpu.SemaphoreType.DMA,
        )

    return sc_kernel


@functools.partial(jax.jit, static_argnums=(4,))
def _run(x_vals, y_vals, write_idx, retrieve_idx, M):
    B, D_IN = x_vals.shape
    D_OUT = y_vals.shape[1]
    sck = _build_sc_kernel(M, B, D_IN, D_OUT)
    return sck(x_vals, y_vals, write_idx, retrieve_idx)


def kernel(buffer_input, buffer_target, x_vals, y_vals, write_idx,
           retrieve_idx):
    M = buffer_input.shape[0]
    ox, oy = _run(x_vals, y_vals,
                  write_idx.astype(jnp.int32),
                  retrieve_idx.astype(jnp.int32), M)
    return (ox, oy)


# speedup vs baseline: 1.3640x; 1.3640x over previous
import functools

import jax
import jax.numpy as jnp
from jax import lax
from jax.experimental import pallas as pl
from jax.experimental.pallas import tpu as pltpu
from jax.experimental.pallas import tpu_sc as plsc

NC = 2
NS = 16
L = 16

CH = 2048


@functools.lru_cache(maxsize=None)
def _build_sc_kernel(M, B, D_IN, D_OUT):
    n_slot = (M + 127) // 128 * 128
    rows_per_core = B // NC
    rows_per_tile = B // (NC * NS)
    n_vidx = rows_per_tile + L

    mesh = plsc.VectorSubcoreMesh(
        core_axis_name="c", subcore_axis_name="s",
        num_cores=NC, num_subcores=NS)

    @functools.partial(
        pl.kernel,
        out_type=(
            jax.ShapeDtypeStruct((B, D_IN), jnp.float32),
            jax.ShapeDtypeStruct((B, D_OUT), jnp.float32),
        ),
        mesh=mesh,
        compiler_params=pltpu.CompilerParams(
            needs_layout_passes=False, use_tc_tiling_on_sc=False),
        scratch_types=[
            pltpu.VMEM_SHARED((rows_per_core,), jnp.int32),
        ],
    )
    def sc_kernel(x_hbm, y_hbm, widx_hbm, ridx_hbm, outx_hbm, outy_hbm,
                  srows_sh):
        cid = lax.axis_index("c")
        sid = lax.axis_index("s")

        @pl.when(sid == 0)
        def _build():
            def build(slot, wbuf, rbuf, robuf):
                iota = jnp.arange(L, dtype=jnp.int32)
                zero16 = jnp.zeros((L,), jnp.int32)
                UNR = 4

                rbase = cid * rows_per_core
                pltpu.sync_copy(
                    ridx_hbm.at[pl.ds(rbase, rows_per_core)], rbuf)

                with jax.named_scope("ph_prezero"):
                    def pz(i, carry):
                        for u in range(UNR):
                            idxv = rbuf[pl.ds((i * UNR + u) * L, L)]
                            plsc.store_scatter(slot, [idxv], zero16)
                        return carry
                    lax.fori_loop(0, rows_per_core // (L * UNR), pz, 0)

                def wchunk(ck, carry):
                    pltpu.sync_copy(widx_hbm.at[pl.ds(ck * CH, CH)], wbuf)

                    def scat(i, c2):
                        for u in range(UNR):
                            idxv = wbuf[pl.ds((i * UNR + u) * L, L)]
                            jv = iota + (ck * CH + (i * UNR + u) * L + 1)
                            plsc.store_scatter(slot, [idxv], jv)
                        return c2
                    lax.fori_loop(0, CH // (L * UNR), scat, 0)

                    def rep(i, c2):
                        for u in range(UNR):
                            idxv = wbuf[pl.ds((i * UNR + u) * L, L)]
                            jv = iota + (ck * CH + (i * UNR + u) * L + 1)
                            m = plsc.load_gather(slot, [idxv]) < jv

                            def wbody(mm):
                                plsc.store_scatter(
                                    slot, [idxv], jv, mask=mm)
                                return plsc.load_gather(slot, [idxv]) < jv
                            lax.while_loop(
                                lambda mm: jnp.any(mm), wbody, m)
                        return c2
                    lax.fori_loop(0, CH // (L * UNR), rep, 0)
                    return carry
                with jax.named_scope("ph_scatter"):
                    lax.fori_loop(0, B // CH, wchunk, 0)

                with jax.named_scope("ph_lookup"):
                    def g(i, c2):
                        for u in range(UNR):
                            idxv = rbuf[pl.ds((i * UNR + u) * L, L)]
                            robuf[pl.ds((i * UNR + u) * L, L)] = (
                                plsc.load_gather(slot, [idxv]))
                        return c2
                    lax.fori_loop(0, rows_per_core // (L * UNR), g, 0)
                    pltpu.sync_copy(robuf, srows_sh)

            pl.run_scoped(
                build,
                pltpu.VMEM((n_slot,), jnp.int32),
                pltpu.VMEM((CH,), jnp.int32),
                pltpu.VMEM((rows_per_core,), jnp.int32),
                pltpu.VMEM((rows_per_core,), jnp.int32),
            )

        plsc.subcore_barrier()

        def phasec(sv, vidx, rnk, xbuf, ybuf, sem):
            base = sid * rows_per_tile
            gbase = cid * rows_per_core + base
            pltpu.sync_copy(srows_sh.at[pl.ds(base, rows_per_tile)], sv)

            zero16 = jnp.zeros((L,), jnp.int32)
            for i in range(n_vidx // L):
                vidx[pl.ds(i * L, L)] = zero16

            def comp(i, off):
                s16 = sv[pl.ds(i * L, L)]
                m = s16 > 0
                mi = jnp.where(m, 1, 0).astype(jnp.int32)
                incl = plsc.cumsum(mi)
                rankv = off + incl - 1
                rnk[pl.ds(i * L, L)] = jnp.where(m, rankv, -1)
                plsc.store_scatter(vidx, [rankv], s16 - 1, mask=m)
                return off + jnp.max(incl)
            off = lax.fori_loop(
                0, rows_per_tile // L, comp,
                jnp.zeros((L,), jnp.int32))
            nv = off[0]

            nch = (nv + L - 1) // L

            def gch(k, carry):
                idxv = vidx[pl.ds(k * L, L)]
                cpx = pltpu.async_copy(
                    x_hbm.at[idxv], xbuf.at[pl.ds(k * L, L)], sem)
                cpy = pltpu.async_copy(
                    y_hbm.at[idxv], ybuf.at[pl.ds(k * L, L)], sem)
                cpx.wait()
                cpy.wait()
                return carry
            with jax.named_scope("ph_gather"):
                lax.fori_loop(0, nch, gch, 0)

            zx = jnp.zeros((L,), jnp.float32)

            def mv(t, carry):
                lp = rows_per_tile - 1 - t
                rv = plsc.load_gather(
                    rnk, [jnp.full((L,), lp, jnp.int32)])[0]

                @pl.when(rv >= 0)
                def _():
                    for v in range(D_IN // L):
                        xbuf[lp, pl.ds(v * L, L)] = xbuf[rv, pl.ds(v * L, L)]
                    for v in range(D_OUT // L):
                        ybuf[lp, pl.ds(v * L, L)] = ybuf[rv, pl.ds(v * L, L)]

                @pl.when(rv < 0)
                def _():
                    for v in range(D_IN // L):
                        xbuf[lp, pl.ds(v * L, L)] = zx
                    for v in range(D_OUT // L):
                        ybuf[lp, pl.ds(v * L, L)] = zx
                return carry
            with jax.named_scope("ph_move"):
                lax.fori_loop(0, rows_per_tile, mv, 0)

            pltpu.sync_copy(xbuf, outx_hbm.at[pl.ds(gbase, rows_per_tile)])
            pltpu.sync_copy(ybuf, outy_hbm.at[pl.ds(gbase, rows_per_tile)])

        pl.run_scoped(
            phasec,
            pltpu.VMEM((rows_per_tile,), jnp.int32),
            pltpu.VMEM((n_vidx,), jnp.int32),
            pltpu.VMEM((rows_per_tile + L,), jnp.int32),
            pltpu.VMEM((rows_per_tile, D_IN), jnp.float32),
            pltpu.VMEM((rows_per_tile, D_OUT), jnp.float32),
            pltpu.SemaphoreType.DMA,
        )

    return sc_kernel


@functools.partial(jax.jit, static_argnums=(4,))
def _run(x_vals, y_vals, write_idx, retrieve_idx, M):
    B, D_IN = x_vals.shape
    D_OUT = y_vals.shape[1]
    sck = _build_sc_kernel(M, B, D_IN, D_OUT)
    return sck(x_vals, y_vals, write_idx, retrieve_idx)


def kernel(buffer_input, buffer_target, x_vals, y_vals, write_idx,
           retrieve_idx):
    M = buffer_input.shape[0]
    ox, oy = _run(x_vals, y_vals,
                  write_idx.astype(jnp.int32),
                  retrieve_idx.astype(jnp.int32), M)
    return (ox, oy)


# speedup vs baseline: 1.3740x; 1.0073x over previous
import functools

import jax
import jax.numpy as jnp
from jax import lax
from jax.experimental import pallas as pl
from jax.experimental.pallas import tpu as pltpu
from jax.experimental.pallas import tpu_sc as plsc

NC = 2
NS = 16
L = 16

CH = 2048


@functools.lru_cache(maxsize=None)
def _build_sc_kernel(M, B, D_IN, D_OUT):
    n_slot = (M + 127) // 128 * 128
    rows_per_core = B // NC
    rows_per_tile = B // (NC * NS)
    n_vidx = rows_per_tile + L

    mesh = plsc.VectorSubcoreMesh(
        core_axis_name="c", subcore_axis_name="s",
        num_cores=NC, num_subcores=NS)

    @functools.partial(
        pl.kernel,
        out_type=(
            jax.ShapeDtypeStruct((B, D_IN), jnp.float32),
            jax.ShapeDtypeStruct((B, D_OUT), jnp.float32),
        ),
        mesh=mesh,
        compiler_params=pltpu.CompilerParams(
            needs_layout_passes=False, use_tc_tiling_on_sc=False),
        scratch_types=[
            pltpu.VMEM_SHARED((rows_per_core,), jnp.int32),
        ],
    )
    def sc_kernel(x_hbm, y_hbm, widx_hbm, ridx_hbm, outx_hbm, outy_hbm,
                  srows_sh):
        cid = lax.axis_index("c")
        sid = lax.axis_index("s")

        @pl.when(sid == 0)
        def _build():
            def build(slot, wbuf, rbuf, robuf):
                iota = jnp.arange(L, dtype=jnp.int32)
                zero16 = jnp.zeros((L,), jnp.int32)
                UNR = 4

                rbase = cid * rows_per_core
                pltpu.sync_copy(
                    ridx_hbm.at[pl.ds(rbase, rows_per_core)], rbuf)

                with jax.named_scope("ph_prezero"):
                    def pz(i, carry):
                        for u in range(UNR):
                            idxv = rbuf[pl.ds((i * UNR + u) * L, L)]
                            plsc.store_scatter(slot, [idxv], zero16)
                        return carry
                    lax.fori_loop(0, rows_per_core // (L * UNR), pz, 0)

                def wchunk(ck, carry):
                    pltpu.sync_copy(widx_hbm.at[pl.ds(ck * CH, CH)], wbuf)

                    def scat(i, c2):
                        for u in range(UNR):
                            idxv = wbuf[pl.ds((i * UNR + u) * L, L)]
                            jv = iota + (ck * CH + (i * UNR + u) * L + 1)
                            plsc.store_scatter(slot, [idxv], jv)
                        return c2
                    lax.fori_loop(0, CH // (L * UNR), scat, 0)

                    def rep(i, c2):
                        for u in range(UNR):
                            idxv = wbuf[pl.ds((i * UNR + u) * L, L)]
                            jv = iota + (ck * CH + (i * UNR + u) * L + 1)
                            m = plsc.load_gather(slot, [idxv]) < jv

                            def wbody(mm):
                                plsc.store_scatter(
                                    slot, [idxv], jv, mask=mm)
                                return plsc.load_gather(slot, [idxv]) < jv
                            lax.while_loop(
                                lambda mm: jnp.any(mm), wbody, m)
                        return c2
                    lax.fori_loop(0, CH // (L * UNR), rep, 0)
                    return carry
                with jax.named_scope("ph_scatter"):
                    lax.fori_loop(0, B // CH, wchunk, 0)

                with jax.named_scope("ph_lookup"):
                    def g(i, c2):
                        for u in range(UNR):
                            idxv = rbuf[pl.ds((i * UNR + u) * L, L)]
                            robuf[pl.ds((i * UNR + u) * L, L)] = (
                                plsc.load_gather(slot, [idxv]))
                        return c2
                    lax.fori_loop(0, rows_per_core // (L * UNR), g, 0)
                    pltpu.sync_copy(robuf, srows_sh)

            pl.run_scoped(
                build,
                pltpu.VMEM((n_slot,), jnp.int32),
                pltpu.VMEM((CH,), jnp.int32),
                pltpu.VMEM((rows_per_core,), jnp.int32),
                pltpu.VMEM((rows_per_core,), jnp.int32),
            )

        plsc.subcore_barrier()

        def phasec(sv, vidx, rnk, xbuf, ybuf, sem):
            base = sid * rows_per_tile
            gbase = cid * rows_per_core + base
            pltpu.sync_copy(srows_sh.at[pl.ds(base, rows_per_tile)], sv)

            zero16 = jnp.zeros((L,), jnp.int32)
            for i in range(n_vidx // L):
                vidx[pl.ds(i * L, L)] = zero16

            def comp(i, off):
                s16 = sv[pl.ds(i * L, L)]
                m = s16 > 0
                mi = jnp.where(m, 1, 0).astype(jnp.int32)
                incl = plsc.cumsum(mi)
                rankv = off + incl - 1
                rnk[pl.ds(i * L, L)] = jnp.where(m, rankv, -1)
                plsc.store_scatter(vidx, [rankv], s16 - 1, mask=m)
                return off + jnp.max(incl)
            off = lax.fori_loop(
                0, rows_per_tile // L, comp,
                jnp.zeros((L,), jnp.int32))
            nv = off[0]

            nch = (nv + L - 1) // L

            def drain(k):
                pltpu.make_async_copy(
                    x_hbm.at[pl.ds(0, L)], xbuf.at[pl.ds(k * L, L)],
                    sem).wait()
                pltpu.make_async_copy(
                    y_hbm.at[pl.ds(0, L)], ybuf.at[pl.ds(k * L, L)],
                    sem).wait()

            def gch(k, carry):
                idxv = vidx[pl.ds(k * L, L)]
                pltpu.async_copy(
                    x_hbm.at[idxv], xbuf.at[pl.ds(k * L, L)], sem)
                pltpu.async_copy(
                    y_hbm.at[idxv], ybuf.at[pl.ds(k * L, L)], sem)

                @pl.when(k > 0)
                def _():
                    drain(k - 1)
                return carry
            with jax.named_scope("ph_gather"):
                lax.fori_loop(0, nch, gch, 0)

                @pl.when(nch > 0)
                def _():
                    drain(nch - 1)

            zx = jnp.zeros((L,), jnp.float32)

            def mv(t, carry):
                i = rows_per_tile // L - 1 - t
                rnkv = rnk[pl.ds(i * L, L)]
                for l in range(L - 1, -1, -1):
                    lp = i * L + l
                    rv = rnkv[l]

                    @pl.when(rv >= 0)
                    def _():
                        for v in range(D_IN // L):
                            xbuf[lp, pl.ds(v * L, L)] = (
                                xbuf[rv, pl.ds(v * L, L)])
                        for v in range(D_OUT // L):
                            ybuf[lp, pl.ds(v * L, L)] = (
                                ybuf[rv, pl.ds(v * L, L)])

                    @pl.when(rv < 0)
                    def _():
                        for v in range(D_IN // L):
                            xbuf[lp, pl.ds(v * L, L)] = zx
                        for v in range(D_OUT // L):
                            ybuf[lp, pl.ds(v * L, L)] = zx
                return carry
            with jax.named_scope("ph_move"):
                lax.fori_loop(0, rows_per_tile // L, mv, 0)

            pltpu.sync_copy(xbuf, outx_hbm.at[pl.ds(gbase, rows_per_tile)])
            pltpu.sync_copy(ybuf, outy_hbm.at[pl.ds(gbase, rows_per_tile)])

        pl.run_scoped(
            phasec,
            pltpu.VMEM((rows_per_tile,), jnp.int32),
            pltpu.VMEM((n_vidx,), jnp.int32),
            pltpu.VMEM((rows_per_tile + L,), jnp.int32),
            pltpu.VMEM((rows_per_tile, D_IN), jnp.float32),
            pltpu.VMEM((rows_per_tile, D_OUT), jnp.float32),
            pltpu.SemaphoreType.DMA,
        )

    return sc_kernel


@functools.partial(jax.jit, static_argnums=(4,))
def _run(x_vals, y_vals, write_idx, retrieve_idx, M):
    B, D_IN = x_vals.shape
    D_OUT = y_vals.shape[1]
    sck = _build_sc_kernel(M, B, D_IN, D_OUT)
    return sck(x_vals, y_vals, write_idx, retrieve_idx)


def kernel(buffer_input, buffer_target, x_vals, y_vals, write_idx,
           retrieve_idx):
    M = buffer_input.shape[0]
    ox, oy = _run(x_vals, y_vals,
                  write_idx.astype(jnp.int32),
                  retrieve_idx.astype(jnp.int32), M)
    return (ox, oy)
